# baseline (speedup 1.0000x reference)
import jax
import jax.numpy as jnp
from jax.experimental import pallas as pl
from jax.experimental.pallas import tpu as pltpu

_DIM = 14
_N_E = 16384
_TEMP = 0.01
_EPS = 1e-5
_BETA = 0.25
_ENTROPY_LOSS_RATIO = 0.1


def _kron7(fp_b, fm_b, base, npix):
    sub_a = jax.lax.broadcasted_iota(jnp.int32, (8, 1), 0)
    a = None
    for d in range(3):
        bit = jnp.bitwise_and(jnp.right_shift(sub_a, d), 1) == 1
        dd = base + d
        f = jnp.where(bit, fp_b[dd:dd + 1, :], fm_b[dd:dd + 1, :])
        a = f if a is None else a * f
    sub_b = jax.lax.broadcasted_iota(jnp.int32, (16, 1), 0)
    bb = None
    for d in range(4):
        bit = jnp.bitwise_and(jnp.right_shift(sub_b, d), 1) == 1
        dd = base + 3 + d
        f = jnp.where(bit, fp_b[dd:dd + 1, :], fm_b[dd:dd + 1, :])
        bb = f if bb is None else bb * f
    return (bb[:, None, :] * a[None, :, :]).reshape(128, npix)


def _lfq_body(z_ref, sign_ref, idx_ref, se_ref, ae_ref, aux_ref, cl_ref,
              acc_ref, part_ref, nsteps, n_tok):
    i = pl.program_id(0)
    z4 = z_ref[...]
    _, _, hh, ww = z4.shape
    npix = hh * ww

    pos4 = z4 > 0.0
    sgn4 = jnp.where(pos4, 1.0, -1.0).astype(jnp.float32)
    sign_ref[...] = sgn4

    z = z4.reshape(_DIM, npix)
    pos = z > 0.0

    d_iota = jax.lax.broadcasted_iota(jnp.int32, z.shape, 0)
    w = jnp.left_shift(jnp.int32(1), d_iota)
    idx_ref[...] = jnp.sum(jnp.where(pos, w, 0), axis=0).reshape(-1)

    scale = jnp.float32(4.0 / _TEMP)
    x = jnp.abs(z) * scale
    u = jnp.exp(-x)
    inv = 1.0 / (1.0 + u)
    big = inv
    small = u * inv

    hb = jnp.log1p(u) + x * small
    se_part = jnp.sum(hb)
    cl_part = jnp.sum((sgn4 - z4) ** 2)

    def _dot_t(a, bb_):
        return jax.lax.dot_general(
            a, bb_, (((1,), (1,)), ((), ())),
            preferred_element_type=jnp.float32)

    fp = jnp.where(pos, big, small)
    fm = jnp.where(pos, small, big)
    lot = _kron7(fp, fm, 0, npix)
    hit = _kron7(fp, fm, 7, npix)
    lh = hit.astype(jnp.bfloat16)
    ll = (hit - lh.astype(jnp.float32)).astype(jnp.bfloat16)
    rh = lot.astype(jnp.bfloat16)
    rl = (lot - rh.astype(jnp.float32)).astype(jnp.bfloat16)
    acc_b = _dot_t(lh, rh) + (_dot_t(lh, rl) + _dot_t(ll, rh))

    @pl.when(i == 0)
    def _():
        acc_ref[...] = acc_b
        part_ref[0] = se_part
        part_ref[1] = cl_part

    @pl.when(i > 0)
    def _():
        acc_ref[...] += acc_b
        part_ref[0] += se_part
        part_ref[1] += cl_part

    @pl.when(i == nsteps - 1)
    def _():
        q = acc_ref[...] * (1.0 / jnp.float32(n_tok))
        ae = -jnp.sum(q * jnp.log(q + jnp.float32(_EPS)))
        se = part_ref[0] / jnp.float32(n_tok)
        cl = part_ref[1] / jnp.float32(n_tok * _DIM)
        se_ref[0] = se
        ae_ref[0] = ae
        aux_ref[0] = jnp.float32(_ENTROPY_LOSS_RATIO) * (se - ae)
        cl_ref[0] = jnp.float32(_BETA) * cl


def kernel(z, codebook):
    del codebook
    b, d, h, w = z.shape
    n_tok = b * h * w

    import functools
    body = functools.partial(_lfq_body, nsteps=b, n_tok=n_tok)
    smem_scalar = pl.BlockSpec((1,), lambda i: (0,), memory_space=pltpu.SMEM)
    q, indices_flat, se, ae, aux, cl = pl.pallas_call(
        body,
        grid=(b,),
        in_specs=(pl.BlockSpec((1, d, h, w), lambda i: (i, 0, 0, 0)),),
        out_shape=(
            jax.ShapeDtypeStruct((b, d, h, w), jnp.float32),
            jax.ShapeDtypeStruct((n_tok,), jnp.int32),
            jax.ShapeDtypeStruct((1,), jnp.float32),
            jax.ShapeDtypeStruct((1,), jnp.float32),
            jax.ShapeDtypeStruct((1,), jnp.float32),
            jax.ShapeDtypeStruct((1,), jnp.float32),
        ),
        out_specs=(
            pl.BlockSpec((1, d, h, w), lambda i: (i, 0, 0, 0)),
            pl.BlockSpec((h * w,), lambda i: (i,)),
            smem_scalar, smem_scalar, smem_scalar, smem_scalar,
        ),
        scratch_shapes=(
            pltpu.VMEM((128, 128), jnp.float32),
            pltpu.SMEM((2,), jnp.float32),
        ),
    )(z)

    return (q, (se[0], ae[0], aux[0], cl[0]), indices_flat)


# speedup vs baseline: 1.1730x; 1.1730x over previous
import jax
import jax.numpy as jnp
from jax.experimental import pallas as pl
from jax.experimental.pallas import tpu as pltpu

_DIM = 14
_N_E = 16384
_TEMP = 0.01
_EPS = 1e-5
_BETA = 0.25
_ENTROPY_LOSS_RATIO = 0.1


def _kron7(fp_b, fm_b, base, npix):
    sub_a = jax.lax.broadcasted_iota(jnp.int32, (8, 1), 0)
    a = None
    for d in range(3):
        bit = jnp.bitwise_and(jnp.right_shift(sub_a, d), 1) == 1
        dd = base + d
        f = jnp.where(bit, fp_b[dd:dd + 1, :], fm_b[dd:dd + 1, :])
        a = f if a is None else a * f
    sub_b = jax.lax.broadcasted_iota(jnp.int32, (16, 1), 0)
    bb = None
    for d in range(4):
        bit = jnp.bitwise_and(jnp.right_shift(sub_b, d), 1) == 1
        dd = base + 3 + d
        f = jnp.where(bit, fp_b[dd:dd + 1, :], fm_b[dd:dd + 1, :])
        bb = f if bb is None else bb * f
    return (bb[:, None, :] * a[None, :, :]).reshape(128, npix)


def _lfq_body(z_ref, sign_ref, idx_ref, se_ref, ae_ref, aux_ref, cl_ref):
    z4 = z_ref[...]
    bsz, _, hh, ww = z4.shape
    npix = hh * ww
    n_tok = bsz * npix

    pos4 = z4 > 0.0
    sgn4 = jnp.where(pos4, 1.0, -1.0).astype(jnp.float32)
    sign_ref[...] = sgn4

    z = z4.reshape(bsz, _DIM, npix)
    pos = z > 0.0

    d_iota = jax.lax.broadcasted_iota(jnp.int32, z.shape, 1)
    w = jnp.left_shift(jnp.int32(1), d_iota)
    idx_ref[...] = jnp.sum(jnp.where(pos, w, 0), axis=1).reshape(-1)

    scale = jnp.float32(4.0 / _TEMP)
    x = jnp.abs(z) * scale
    u = jnp.exp(-x)
    inv = 1.0 / (1.0 + u)
    big = inv
    small = u * inv

    hb = jnp.log1p(u) + x * small
    se = jnp.sum(hb) / jnp.float32(n_tok)

    cl = jnp.sum((sgn4 - z4) ** 2) / jnp.float32(n_tok * _DIM)

    def _dot_t(a, bb_):
        return jax.lax.dot_general(
            a, bb_, (((1,), (1,)), ((), ())),
            preferred_element_type=jnp.float32)

    fp = jnp.where(pos, big, small)
    fm = jnp.where(pos, small, big)
    acc = jnp.zeros((128, 128), jnp.float32)
    for b in range(bsz):
        lot = _kron7(fp[b], fm[b], 0, npix)
        hit = _kron7(fp[b], fm[b], 7, npix)
        lh = hit.astype(jnp.bfloat16)
        ll = (hit - lh.astype(jnp.float32)).astype(jnp.bfloat16)
        rh = lot.astype(jnp.bfloat16)
        rl = (lot - rh.astype(jnp.float32)).astype(jnp.bfloat16)
        acc = acc + (_dot_t(lh, rh) + (_dot_t(lh, rl) + _dot_t(ll, rh)))

    q = acc * (1.0 / jnp.float32(n_tok))
    ae = -jnp.sum(q * jnp.log(q + jnp.float32(_EPS)))

    se_ref[...] = se
    ae_ref[...] = ae
    aux_ref[...] = jnp.float32(_ENTROPY_LOSS_RATIO) * (se - ae)
    cl_ref[...] = jnp.float32(_BETA) * cl


def kernel(z, codebook):
    del codebook
    b, d, h, w = z.shape

    smem_scalar = pl.BlockSpec(memory_space=pltpu.SMEM)
    q, indices_flat, se, ae, aux, cl = pl.pallas_call(
        _lfq_body,
        out_shape=(
            jax.ShapeDtypeStruct((b, d, h, w), jnp.float32),
            jax.ShapeDtypeStruct((b * h * w,), jnp.int32),
            jax.ShapeDtypeStruct((), jnp.float32),
            jax.ShapeDtypeStruct((), jnp.float32),
            jax.ShapeDtypeStruct((), jnp.float32),
            jax.ShapeDtypeStruct((), jnp.float32),
        ),
        out_specs=(
            pl.BlockSpec(memory_space=pltpu.VMEM),
            pl.BlockSpec(memory_space=pltpu.VMEM),
            smem_scalar, smem_scalar, smem_scalar, smem_scalar,
        ),
    )(z)

    return (q, (se, ae, aux, cl), indices_flat)


# speedup vs baseline: 1.2085x; 1.0302x over previous
import jax
import jax.numpy as jnp
from jax.experimental import pallas as pl
from jax.experimental.pallas import tpu as pltpu

_DIM = 14
_N_E = 16384
_TEMP = 0.01
_EPS = 1e-5
_BETA = 0.25
_ENTROPY_LOSS_RATIO = 0.1


def _kron7(fp_b, fm_b, base, npix):
    sub_a = jax.lax.broadcasted_iota(jnp.int32, (8, 1), 0)
    a = None
    for d in range(3):
        bit = jnp.bitwise_and(jnp.right_shift(sub_a, d), 1) == 1
        dd = base + d
        f = jnp.where(bit, fp_b[dd:dd + 1, :], fm_b[dd:dd + 1, :])
        a = f if a is None else a * f
    sub_b = jax.lax.broadcasted_iota(jnp.int32, (16, 1), 0)
    bb = None
    for d in range(4):
        bit = jnp.bitwise_and(jnp.right_shift(sub_b, d), 1) == 1
        dd = base + 3 + d
        f = jnp.where(bit, fp_b[dd:dd + 1, :], fm_b[dd:dd + 1, :])
        bb = f if bb is None else bb * f
    return (bb[:, None, :] * a[None, :, :]).reshape(128, npix)


def _lfq_body(z_ref, sign_ref, idx_ref, se_ref, ae_ref, aux_ref, cl_ref):
    z4 = z_ref[...]
    bsz, _, hh, ww = z4.shape
    npix = hh * ww
    n_tok = bsz * npix

    pos4 = z4 > 0.0
    sgn4 = jnp.where(pos4, 1.0, -1.0).astype(jnp.float32)
    sign_ref[...] = sgn4

    z = z4.reshape(bsz, _DIM, npix)
    pos = z > 0.0

    d_iota = jax.lax.broadcasted_iota(jnp.int32, z.shape, 1)
    w = jnp.left_shift(jnp.int32(1), d_iota)
    idx_ref[...] = jnp.sum(jnp.where(pos, w, 0), axis=1).reshape(-1)

    scale = jnp.float32(4.0 / _TEMP)
    az = jnp.abs(z)
    x = az * scale
    u = jnp.exp(-x)
    inv = 1.0 / (1.0 + u)
    big = inv
    small = u * inv

    hb = jnp.log1p(u) + x * small
    se = jnp.sum(hb) / jnp.float32(n_tok)

    cl = jnp.sum((1.0 - az) ** 2) / jnp.float32(n_tok * _DIM)

    def _dot_t(a, bb_):
        return jax.lax.dot_general(
            a, bb_, (((1,), (1,)), ((), ())),
            preferred_element_type=jnp.float32)

    fp = jnp.where(pos, big, small)
    fm = jnp.where(pos, small, big)
    acc = jnp.zeros((128, 128), jnp.float32)
    for b in range(bsz):
        lot = _kron7(fp[b], fm[b], 0, npix)
        hit = _kron7(fp[b], fm[b], 7, npix)
        lh = hit.astype(jnp.bfloat16)
        ll = (hit - lh.astype(jnp.float32)).astype(jnp.bfloat16)
        rh = lot.astype(jnp.bfloat16)
        rl = (lot - rh.astype(jnp.float32)).astype(jnp.bfloat16)
        acc = acc + (_dot_t(lh, rh) + (_dot_t(lh, rl) + _dot_t(ll, rh)))

    q = acc * (1.0 / jnp.float32(n_tok))
    ae = -jnp.sum(q * jnp.log(q + jnp.float32(_EPS)))

    se_ref[...] = se
    ae_ref[...] = ae
    aux_ref[...] = jnp.float32(_ENTROPY_LOSS_RATIO) * (se - ae)
    cl_ref[...] = jnp.float32(_BETA) * cl


def kernel(z, codebook):
    del codebook
    b, d, h, w = z.shape

    smem_scalar = pl.BlockSpec(memory_space=pltpu.SMEM)
    q, indices_flat, se, ae, aux, cl = pl.pallas_call(
        _lfq_body,
        out_shape=(
            jax.ShapeDtypeStruct((b, d, h, w), jnp.float32),
            jax.ShapeDtypeStruct((b * h * w,), jnp.int32),
            jax.ShapeDtypeStruct((), jnp.float32),
            jax.ShapeDtypeStruct((), jnp.float32),
            jax.ShapeDtypeStruct((), jnp.float32),
            jax.ShapeDtypeStruct((), jnp.float32),
        ),
        out_specs=(
            pl.BlockSpec(memory_space=pltpu.VMEM),
            pl.BlockSpec(memory_space=pltpu.VMEM),
            smem_scalar, smem_scalar, smem_scalar, smem_scalar,
        ),
    )(z)

    return (q, (se, ae, aux, cl), indices_flat)
